# baseline (speedup 1.0000x reference)
import functools

import jax
import jax.numpy as jnp
from jax import lax
from jax.experimental import pallas as pl
from jax.experimental.pallas import tpu as pltpu
from jax.experimental.pallas import tpu_sc as plsc

_B = 16384
_D = 64
_NE = 1000000
_NW = 32
_W = 512
_WSH = 9
_NFULL = _NE // _W
_TAILC0 = _NFULL * _W
_TAILW = _NE - _TAILC0
_TAIL_TILE = _NFULL % _NW
_CAP = 8192
_NBLK = 8
_CAPB = 1024
_CAPBP = _CAPB + 16
_RING = 4
_CHUNK = 4096

_SENT = 0x7FFFFFF

_mesh = plsc.VectorSubcoreMesh(core_axis_name="c", subcore_axis_name="s")
_params = pltpu.CompilerParams(use_tc_tiling_on_sc=True,
                               needs_layout_passes=False)


def _flush_group(win_par, gb_idx, gb_row, ring, scratch, ssem, grp, nvalid):
    lanes = lax.iota(jnp.int32, 16)
    gi = gb_idx[pl.ds(0, 16)]
    gr = gb_row[pl.ds(0, 16)]
    mv = lanes < nvalid
    gi = jnp.where(mv, gi, jnp.broadcast_to(gi[0], (16,)))
    gr = jnp.where(mv, gr, jnp.broadcast_to(gr[0], (16,)))
    slot = lax.rem(grp, jnp.int32(_RING))

    @pl.when(grp >= _RING)
    def _():
        pltpu.make_async_copy(scratch.at[pl.ds(0, 16)], ring.at[0], ssem).wait()

    d0s = [2 * q + lanes // 8 for q in range(4)]
    d1 = lanes % 8
    for item in range(16):
        e16 = jnp.broadcast_to(gi[item], (16,))
        for q in range(4):
            fv = plsc.load_gather(win_par, [d0s[q], d1, e16])
            ring[slot, item, pl.ds(16 * q, 16)] = fv
    pltpu.async_copy(ring.at[slot], scratch.at[gr], ssem)
    return grp + 1


def _make_window_processor(gb_idx, gb_row, blk_idx, blk_row, ring,
                           scratch, ssem):
    def process(w, c0, base, nvecb, win_par, grp):
        def vbody(j, carry):
            grp, bc = carry
            iv = blk_idx[pl.ds(base + j * 16, 16)]
            m = lax.shift_right_logical(iv, _WSH) == w
            nm = jnp.sum(m.astype(jnp.int32))

            def append(carry):
                grp, bc = carry
                rv = blk_row[pl.ds(base + j * 16, 16)]
                plsc.store_compressed(gb_idx.at[pl.ds(bc, 16)], iv - c0, mask=m)
                plsc.store_compressed(gb_row.at[pl.ds(bc, 16)], rv, mask=m)
                bc = bc + nm

                def full_flush(carry):
                    grp, bc = carry
                    grp = _flush_group(win_par, gb_idx, gb_row, ring, scratch,
                                       ssem, grp, jnp.int32(16))
                    gb_idx[pl.ds(0, 16)] = gb_idx[pl.ds(16, 16)]
                    gb_row[pl.ds(0, 16)] = gb_row[pl.ds(16, 16)]
                    return (grp, bc - 16)

                return lax.cond(bc >= 16, full_flush, lambda c: c, (grp, bc))

            return lax.cond(nm > 0, append, lambda c: c, (grp, bc))

        grp, bc = lax.fori_loop(0, nvecb, vbody, (grp, jnp.int32(0)))

        def tail_flush(carry):
            grp, bc = carry
            return (_flush_group(win_par, gb_idx, gb_row, ring, scratch,
                                 ssem, grp, bc), jnp.int32(0))

        grp, _ = lax.cond(bc > 0, tail_flush, lambda c: c, (grp, bc))
        return grp

    return process


@functools.partial(
    pl.kernel,
    mesh=_mesh,
    compiler_params=_params,
    out_type=[jax.ShapeDtypeStruct((2 * _B, 128), jnp.float32)],
    scratch_types=[
        pltpu.VMEM((_CHUNK,), jnp.int32),
        pltpu.VMEM((_CAP + 16,), jnp.int32),
        pltpu.VMEM((_CAP + 16,), jnp.int32),
        pltpu.VMEM((_NBLK * _CAPBP,), jnp.int32),
        pltpu.VMEM((_NBLK * _CAPBP,), jnp.int32),
        pltpu.VMEM((48,), jnp.int32),
        pltpu.VMEM((48,), jnp.int32),
        pltpu.VMEM((2, 8, 8, _W), jnp.float32),
        pltpu.VMEM((_RING, 16, 128), jnp.float32),
        pltpu.SMEM((_NBLK,), jnp.int32),
        pltpu.SemaphoreType.DMA,
        pltpu.SemaphoreType.DMA,
    ],
)
def _scan_kernel(head_hbm, tail_hbm, entT_hbm, tailblk_hbm, scratch_hbm,
                 idx_buf, lst_idx, lst_row, blk_idx, blk_row, gb_idx, gb_row,
                 win, ring, blk_n, wsem, ssem):
    wid = lax.axis_index("s") * 2 + lax.axis_index("c")
    lanes = lax.iota(jnp.int32, 16)

    def scan_half(src_hbm, sel, cnt0):
        def chunk_body(c, cnt):
            pltpu.sync_copy(src_hbm.at[pl.ds(c * _CHUNK, _CHUNK)], idx_buf)

            def vbody(j, cnt):
                iv = idx_buf[pl.ds(j * 16, 16)]
                mine = (lax.shift_right_logical(iv, _WSH) & (_NW - 1)) == wid
                rowid = lanes + (c * _CHUNK + j * 16 + sel * _B)
                plsc.store_compressed(lst_idx.at[pl.ds(cnt, 16)], iv, mask=mine)
                plsc.store_compressed(lst_row.at[pl.ds(cnt, 16)], rowid,
                                      mask=mine)
                return jnp.minimum(cnt + jnp.sum(mine.astype(jnp.int32)),
                                   jnp.int32(_CAP - 16))

            return lax.fori_loop(0, _CHUNK // 16, vbody, cnt)

        return lax.fori_loop(0, _B // _CHUNK, chunk_body, cnt0)

    cnt = scan_half(head_hbm, 0, jnp.int32(0))
    cnt = scan_half(tail_hbm, 1, cnt)
    lst_idx[pl.ds(cnt, 16)] = jnp.full((16,), _SENT, jnp.int32)
    lst_row[pl.ds(cnt, 16)] = jnp.zeros((16,), jnp.int32)
    nvec = lax.div(cnt + 15, jnp.int32(16))

    def a2body(j, counts):
        iv = lst_idx[pl.ds(j * 16, 16)]
        rv = lst_row[pl.ds(j * 16, 16)]
        kk = lax.shift_right_logical(
            lax.shift_right_logical(iv, _WSH) - wid, 5)
        blk = lax.shift_right_logical(kk, 3)
        out = []
        for b in range(_NBLK):
            m = blk == b
            plsc.store_compressed(
                blk_idx.at[pl.ds(b * _CAPBP + counts[b], 16)], iv, mask=m)
            plsc.store_compressed(
                blk_row.at[pl.ds(b * _CAPBP + counts[b], 16)], rv, mask=m)
            out.append(jnp.minimum(counts[b] + jnp.sum(m.astype(jnp.int32)),
                                   jnp.int32(_CAPB - 16)))
        return tuple(out)

    counts = lax.fori_loop(0, nvec, a2body, tuple(jnp.int32(0)
                                                  for _ in range(_NBLK)))
    for b in range(_NBLK):
        blk_idx[pl.ds(b * _CAPBP + counts[b], 16)] = jnp.full((16,), _SENT,
                                                              jnp.int32)
        blk_n[b] = lax.div(counts[b] + 15, jnp.int32(16))

    process = _make_window_processor(gb_idx, gb_row, blk_idx, blk_row, ring,
                                     scratch_hbm, ssem)

    nwin_mine = lax.div(jnp.int32(_NFULL - 1 + _NW) - wid, jnp.int32(_NW))

    def issue(k, par):
        w = wid + _NW * k
        c0 = pl.multiple_of(w * _W, _W)
        for g in range(8):
            pltpu.async_copy(entT_hbm.at[pl.ds(8 * g, 8), pl.ds(c0, _W)],
                             win.at[par, g], wsem)

    def drain_win(par):
        for g in range(8):
            pltpu.make_async_copy(entT_hbm.at[pl.ds(0, 8), pl.ds(0, _W)],
                                  win.at[par, g], wsem).wait()

    issue(jnp.int32(0), jnp.int32(0))

    def wbody(k, grp):
        par = lax.rem(k, jnp.int32(2))
        drain_win(par)
        kn = jnp.minimum(k + 1, nwin_mine - 1)
        issue(kn, 1 - par)
        w = wid + _NW * k
        c0 = w * _W
        b = lax.shift_right_logical(k, 3)
        grp = process(w, c0, b * _CAPBP, blk_n[b], win.at[par], grp)
        return grp

    grp = lax.fori_loop(0, nwin_mine, wbody, jnp.int32(0))
    drain_win(lax.rem(nwin_mine, jnp.int32(2)))

    @pl.when(wid == _TAIL_TILE)
    def _():
        for g in range(8):
            pltpu.async_copy(tailblk_hbm.at[pl.ds(8 * g, 8)],
                             win.at[0, g, slice(None), pl.ds(0, 128)], wsem)
        for g in range(8):
            pltpu.make_async_copy(tailblk_hbm.at[pl.ds(0, 8)],
                                  win.at[0, g, slice(None), pl.ds(0, 128)],
                                  wsem).wait()
        ktail = lax.div(jnp.int32(_NFULL) - wid, jnp.int32(_NW))
        bt = lax.shift_right_logical(ktail, 3)
        g2 = process(jnp.int32(_NFULL), jnp.int32(_TAILC0), bt * _CAPBP,
                     blk_n[bt], win.at[0], grp)

        def drain(i, x):
            pltpu.make_async_copy(scratch_hbm.at[pl.ds(0, 16)], ring.at[0],
                                  ssem).wait()
            return x
        lax.fori_loop(0, jnp.minimum(g2, jnp.int32(_RING)), drain,
                      jnp.int32(0))

    @pl.when(wid != _TAIL_TILE)
    def _():
        def drain(i, x):
            pltpu.make_async_copy(scratch_hbm.at[pl.ds(0, 16)], ring.at[0],
                                  ssem).wait()
            return x
        lax.fori_loop(0, jnp.minimum(grp, jnp.int32(_RING)), drain,
                      jnp.int32(0))


@functools.partial(
    pl.kernel,
    mesh=_mesh,
    compiler_params=_params,
    out_type=[
        jax.ShapeDtypeStruct((_D, _B), jnp.float32),
        jax.ShapeDtypeStruct((_D, _B), jnp.float32),
        jax.ShapeDtypeStruct((_D, _B), jnp.float32),
    ],
    scratch_types=[
        pltpu.VMEM((512,), jnp.int32),
        pltpu.VMEM((2, 128, 128), jnp.float32),
        pltpu.VMEM((2, _D, 128), jnp.float32),
        pltpu.VMEM((16, 17), jnp.float32),
        pltpu.SemaphoreType.DMA,
        pltpu.SemaphoreType.DMA,
    ],
)
def _final_kernel(scratch_hbm, relp_hbm, rel_hbm, out_h, out_r, out_t,
                  relidx, src, tbuf, bb, rsem, osem):
    wid = lax.axis_index("s") * 2 + lax.axis_index("c")
    lanes = lax.iota(jnp.int32, 16)
    b0 = pl.multiple_of(wid * 512, 128)
    pltpu.sync_copy(rel_hbm.at[pl.ds(b0, 512)], relidx)

    steps = [(kind, c) for c in range(4) for kind in ("h", "t", "r")]

    def issue_read(s, par):
        kind, c = steps[s]
        bc = pl.multiple_of(wid * 512 + c * 128, 128)
        if kind == "h":
            pltpu.async_copy(scratch_hbm.at[pl.ds(bc, 128)], src.at[par], rsem)
        elif kind == "t":
            pltpu.async_copy(scratch_hbm.at[pl.ds(_B + bc, 128)], src.at[par],
                             rsem)
        else:
            pltpu.async_copy(relp_hbm.at[relidx.at[pl.ds(c * 128, 128)]],
                             src.at[par], rsem)

    issue_read(0, 0)
    for s, (kind, c) in enumerate(steps):
        par = s & 1
        pltpu.make_async_copy(scratch_hbm.at[pl.ds(0, 128)], src.at[par],
                              rsem).wait()
        if s + 1 < len(steps):
            issue_read(s + 1, 1 - par)
        if s >= 2:
            pltpu.make_async_copy(tbuf.at[par],
                                  out_h.at[slice(None), pl.ds(0, 128)],
                                  osem).wait()

        def jbody(jb, x):
            for fb in range(4):
                for r in range(16):
                    bb[r, pl.ds(0, 16)] = src[par, jb * 16 + r,
                                              pl.ds(16 * fb, 16)]
                for fo in range(16):
                    col = plsc.load_gather(
                        bb, [lanes, jnp.broadcast_to(jnp.int32(fo), (16,))])
                    tbuf[par, 16 * fb + fo, pl.ds(jb * 16, 16)] = col
            return x

        lax.fori_loop(0, 8, jbody, jnp.int32(0))
        bc = pl.multiple_of(wid * 512 + c * 128, 128)
        out_ref = {"h": out_h, "t": out_t, "r": out_r}[kind]
        pltpu.async_copy(tbuf.at[par], out_ref.at[slice(None), pl.ds(bc, 128)],
                         osem)
    for _ in range(2):
        pltpu.make_async_copy(tbuf.at[0],
                              out_h.at[slice(None), pl.ds(0, 128)],
                              osem).wait()


def kernel(head, relation, tail, entity_table, relation_table):
    head = head.astype(jnp.int32)
    relation = relation.astype(jnp.int32)
    tail = tail.astype(jnp.int32)
    ent_t = jnp.transpose(entity_table)
    relp = jnp.pad(relation_table, ((0, 0), (0, 64)))
    tailblk = jnp.pad(ent_t[:, _TAILC0:], ((0, 0), (0, 128 - _TAILW)))
    (scratch,) = _scan_kernel(head, tail, ent_t, tailblk)
    out_h, out_r, out_t = _final_kernel(scratch, relp, relation)
    return (jnp.transpose(out_h), jnp.transpose(out_r), jnp.transpose(out_t))


# speedup vs baseline: 1.0554x; 1.0554x over previous
import functools

import jax
import jax.numpy as jnp
from jax import lax
from jax.experimental import pallas as pl
from jax.experimental.pallas import tpu as pltpu
from jax.experimental.pallas import tpu_sc as plsc

_B = 16384
_D = 64
_NE = 1000000
_NW = 32
_W = 512
_WSH = 9
_NFULL = _NE // _W
_TAILC0 = _NFULL * _W
_TAILW = _NE - _TAILC0
_TAIL_TILE = _NFULL % _NW
_CAP = 8192
_NBLK = 8
_CAPB = 1024
_CAPBP = _CAPB + 16
_RING = 4
_CHUNK = 4096

_SENT = 0x7FFFFFF

_mesh = plsc.VectorSubcoreMesh(core_axis_name="c", subcore_axis_name="s")
_params = pltpu.CompilerParams(use_tc_tiling_on_sc=True,
                               needs_layout_passes=False)


def _flush_group(win_par, gb_idx, gb_row, ring, scratch, ssem, grp, nvalid):
    lanes = lax.iota(jnp.int32, 16)
    gi = gb_idx[pl.ds(0, 16)]
    gr = gb_row[pl.ds(0, 16)]
    mv = lanes < nvalid
    gi = jnp.where(mv, gi, jnp.broadcast_to(gi[0], (16,)))
    gr = jnp.where(mv, gr, jnp.broadcast_to(gr[0], (16,)))
    slot = lax.rem(grp, jnp.int32(_RING))

    @pl.when(grp >= _RING)
    def _():
        pltpu.make_async_copy(scratch.at[pl.ds(0, 16)], ring.at[0], ssem).wait()

    d0s = [2 * q + lanes // 8 for q in range(4)]
    d1 = lanes % 8
    for item in range(16):
        e16 = jnp.broadcast_to(gi[item], (16,))
        for q in range(4):
            fv = plsc.load_gather(win_par, [d0s[q], d1, e16])
            ring[slot, item, pl.ds(16 * q, 16)] = fv
    pltpu.async_copy(ring.at[slot], scratch.at[gr], ssem)
    return grp + 1


def _make_window_processor(gb_idx, gb_row, blk_idx, blk_row, ring,
                           scratch, ssem):
    def process(w, c0, base, nvecb, win_par, grp):
        def vbody(j, carry):
            grp, bc = carry
            iv = blk_idx[pl.ds(base + j * 16, 16)]
            m = lax.shift_right_logical(iv, _WSH) == w
            nm = jnp.sum(m.astype(jnp.int32))

            def append(carry):
                grp, bc = carry
                rv = blk_row[pl.ds(base + j * 16, 16)]
                plsc.store_compressed(gb_idx.at[pl.ds(bc, 16)], iv - c0, mask=m)
                plsc.store_compressed(gb_row.at[pl.ds(bc, 16)], rv, mask=m)
                bc = bc + nm

                def full_flush(carry):
                    grp, bc = carry
                    grp = _flush_group(win_par, gb_idx, gb_row, ring, scratch,
                                       ssem, grp, jnp.int32(16))
                    gb_idx[pl.ds(0, 16)] = gb_idx[pl.ds(16, 16)]
                    gb_row[pl.ds(0, 16)] = gb_row[pl.ds(16, 16)]
                    return (grp, bc - 16)

                return lax.cond(bc >= 16, full_flush, lambda c: c, (grp, bc))

            return lax.cond(nm > 0, append, lambda c: c, (grp, bc))

        grp, bc = lax.fori_loop(0, nvecb, vbody, (grp, jnp.int32(0)))

        def tail_flush(carry):
            grp, bc = carry
            return (_flush_group(win_par, gb_idx, gb_row, ring, scratch,
                                 ssem, grp, bc), jnp.int32(0))

        grp, _ = lax.cond(bc > 0, tail_flush, lambda c: c, (grp, bc))
        return grp

    return process


@functools.partial(
    pl.kernel,
    mesh=_mesh,
    compiler_params=_params,
    out_type=[jax.ShapeDtypeStruct((2 * _B, 128), jnp.float32)],
    scratch_types=[
        pltpu.VMEM((_CHUNK,), jnp.int32),
        pltpu.VMEM((_CAP + 16,), jnp.int32),
        pltpu.VMEM((_CAP + 16,), jnp.int32),
        pltpu.VMEM((_NBLK * _CAPBP,), jnp.int32),
        pltpu.VMEM((_NBLK * _CAPBP,), jnp.int32),
        pltpu.VMEM((48,), jnp.int32),
        pltpu.VMEM((48,), jnp.int32),
        pltpu.VMEM((2, 8, 8, _W + 1), jnp.float32),
        pltpu.VMEM((_RING, 16, 128), jnp.float32),
        pltpu.SMEM((_NBLK,), jnp.int32),
        pltpu.SemaphoreType.DMA,
        pltpu.SemaphoreType.DMA,
    ],
)
def _scan_kernel(head_hbm, tail_hbm, entT_hbm, tailblk_hbm, scratch_hbm,
                 idx_buf, lst_idx, lst_row, blk_idx, blk_row, gb_idx, gb_row,
                 win, ring, blk_n, wsem, ssem):
    wid = lax.axis_index("s") * 2 + lax.axis_index("c")
    lanes = lax.iota(jnp.int32, 16)

    def scan_half(src_hbm, sel, cnt0):
        def chunk_body(c, cnt):
            pltpu.sync_copy(src_hbm.at[pl.ds(c * _CHUNK, _CHUNK)], idx_buf)

            def vbody(j, cnt):
                iv = idx_buf[pl.ds(j * 16, 16)]
                mine = (lax.shift_right_logical(iv, _WSH) & (_NW - 1)) == wid
                rowid = lanes + (c * _CHUNK + j * 16 + sel * _B)
                plsc.store_compressed(lst_idx.at[pl.ds(cnt, 16)], iv, mask=mine)
                plsc.store_compressed(lst_row.at[pl.ds(cnt, 16)], rowid,
                                      mask=mine)
                return jnp.minimum(cnt + jnp.sum(mine.astype(jnp.int32)),
                                   jnp.int32(_CAP - 16))

            return lax.fori_loop(0, _CHUNK // 16, vbody, cnt)

        return lax.fori_loop(0, _B // _CHUNK, chunk_body, cnt0)

    cnt = scan_half(head_hbm, 0, jnp.int32(0))
    cnt = scan_half(tail_hbm, 1, cnt)
    lst_idx[pl.ds(cnt, 16)] = jnp.full((16,), _SENT, jnp.int32)
    lst_row[pl.ds(cnt, 16)] = jnp.zeros((16,), jnp.int32)
    nvec = lax.div(cnt + 15, jnp.int32(16))

    def a2body(j, counts):
        iv = lst_idx[pl.ds(j * 16, 16)]
        rv = lst_row[pl.ds(j * 16, 16)]
        kk = lax.shift_right_logical(
            lax.shift_right_logical(iv, _WSH) - wid, 5)
        blk = lax.shift_right_logical(kk, 3)
        out = []
        for b in range(_NBLK):
            m = blk == b
            plsc.store_compressed(
                blk_idx.at[pl.ds(b * _CAPBP + counts[b], 16)], iv, mask=m)
            plsc.store_compressed(
                blk_row.at[pl.ds(b * _CAPBP + counts[b], 16)], rv, mask=m)
            out.append(jnp.minimum(counts[b] + jnp.sum(m.astype(jnp.int32)),
                                   jnp.int32(_CAPB - 16)))
        return tuple(out)

    counts = lax.fori_loop(0, nvec, a2body, tuple(jnp.int32(0)
                                                  for _ in range(_NBLK)))
    for b in range(_NBLK):
        blk_idx[pl.ds(b * _CAPBP + counts[b], 16)] = jnp.full((16,), _SENT,
                                                              jnp.int32)
        blk_n[b] = lax.div(counts[b] + 15, jnp.int32(16))

    process = _make_window_processor(gb_idx, gb_row, blk_idx, blk_row, ring,
                                     scratch_hbm, ssem)

    nwin_mine = lax.div(jnp.int32(_NFULL - 1 + _NW) - wid, jnp.int32(_NW))

    def issue(k, par):
        w = wid + _NW * k
        c0 = pl.multiple_of(w * _W, _W)
        for g in range(8):
            pltpu.async_copy(entT_hbm.at[pl.ds(8 * g, 8), pl.ds(c0, _W)],
                             win.at[par, g, slice(None), pl.ds(0, _W)], wsem)

    def drain_win(par):
        for g in range(8):
            pltpu.make_async_copy(entT_hbm.at[pl.ds(0, 8), pl.ds(0, _W)],
                                  win.at[par, g, slice(None), pl.ds(0, _W)],
                                  wsem).wait()

    issue(jnp.int32(0), jnp.int32(0))

    def wbody(k, grp):
        par = lax.rem(k, jnp.int32(2))
        drain_win(par)
        kn = jnp.minimum(k + 1, nwin_mine - 1)
        issue(kn, 1 - par)
        w = wid + _NW * k
        c0 = w * _W
        b = lax.shift_right_logical(k, 3)
        grp = process(w, c0, b * _CAPBP, blk_n[b], win.at[par], grp)
        return grp

    grp = lax.fori_loop(0, nwin_mine, wbody, jnp.int32(0))
    drain_win(lax.rem(nwin_mine, jnp.int32(2)))

    @pl.when(wid == _TAIL_TILE)
    def _():
        for g in range(8):
            pltpu.async_copy(tailblk_hbm.at[pl.ds(8 * g, 8)],
                             win.at[0, g, slice(None), pl.ds(0, 128)], wsem)
        for g in range(8):
            pltpu.make_async_copy(tailblk_hbm.at[pl.ds(0, 8)],
                                  win.at[0, g, slice(None), pl.ds(0, 128)],
                                  wsem).wait()
        ktail = lax.div(jnp.int32(_NFULL) - wid, jnp.int32(_NW))
        bt = lax.shift_right_logical(ktail, 3)
        g2 = process(jnp.int32(_NFULL), jnp.int32(_TAILC0), bt * _CAPBP,
                     blk_n[bt], win.at[0], grp)

        def drain(i, x):
            pltpu.make_async_copy(scratch_hbm.at[pl.ds(0, 16)], ring.at[0],
                                  ssem).wait()
            return x
        lax.fori_loop(0, jnp.minimum(g2, jnp.int32(_RING)), drain,
                      jnp.int32(0))

    @pl.when(wid != _TAIL_TILE)
    def _():
        def drain(i, x):
            pltpu.make_async_copy(scratch_hbm.at[pl.ds(0, 16)], ring.at[0],
                                  ssem).wait()
            return x
        lax.fori_loop(0, jnp.minimum(grp, jnp.int32(_RING)), drain,
                      jnp.int32(0))


@functools.partial(
    pl.kernel,
    mesh=_mesh,
    compiler_params=_params,
    out_type=[
        jax.ShapeDtypeStruct((_D, _B), jnp.float32),
        jax.ShapeDtypeStruct((_D, _B), jnp.float32),
        jax.ShapeDtypeStruct((_D, _B), jnp.float32),
    ],
    scratch_types=[
        pltpu.VMEM((512,), jnp.int32),
        pltpu.VMEM((2, 128, 128), jnp.float32),
        pltpu.VMEM((2, _D, 128), jnp.float32),
        pltpu.VMEM((16, 17), jnp.float32),
        pltpu.SemaphoreType.DMA,
        pltpu.SemaphoreType.DMA,
    ],
)
def _final_kernel(scratch_hbm, relp_hbm, rel_hbm, out_h, out_r, out_t,
                  relidx, src, tbuf, bb, rsem, osem):
    wid = lax.axis_index("s") * 2 + lax.axis_index("c")
    lanes = lax.iota(jnp.int32, 16)
    b0 = pl.multiple_of(wid * 512, 128)
    pltpu.sync_copy(rel_hbm.at[pl.ds(b0, 512)], relidx)

    steps = [(kind, c) for c in range(4) for kind in ("h", "t", "r")]

    def issue_read(s, par):
        kind, c = steps[s]
        bc = pl.multiple_of(wid * 512 + c * 128, 128)
        if kind == "h":
            pltpu.async_copy(scratch_hbm.at[pl.ds(bc, 128)], src.at[par], rsem)
        elif kind == "t":
            pltpu.async_copy(scratch_hbm.at[pl.ds(_B + bc, 128)], src.at[par],
                             rsem)
        else:
            pltpu.async_copy(relp_hbm.at[relidx.at[pl.ds(c * 128, 128)]],
                             src.at[par], rsem)

    issue_read(0, 0)
    for s, (kind, c) in enumerate(steps):
        par = s & 1
        pltpu.make_async_copy(scratch_hbm.at[pl.ds(0, 128)], src.at[par],
                              rsem).wait()
        if s + 1 < len(steps):
            issue_read(s + 1, 1 - par)
        if s >= 2:
            pltpu.make_async_copy(tbuf.at[par],
                                  out_h.at[slice(None), pl.ds(0, 128)],
                                  osem).wait()

        def fbody(f8, x):
            for fo in range(8):
                f = f8 * 8 + fo
                f16 = jnp.broadcast_to(f, (16,))
                for blk in range(8):
                    fv = plsc.load_gather(src.at[par],
                                          [16 * blk + lanes, f16])
                    tbuf[par, f, pl.ds(16 * blk, 16)] = fv
            return x

        lax.fori_loop(0, 8, fbody, jnp.int32(0))
        bc = pl.multiple_of(wid * 512 + c * 128, 128)
        out_ref = {"h": out_h, "t": out_t, "r": out_r}[kind]
        pltpu.async_copy(tbuf.at[par], out_ref.at[slice(None), pl.ds(bc, 128)],
                         osem)
    for _ in range(2):
        pltpu.make_async_copy(tbuf.at[0],
                              out_h.at[slice(None), pl.ds(0, 128)],
                              osem).wait()


def kernel(head, relation, tail, entity_table, relation_table):
    head = head.astype(jnp.int32)
    relation = relation.astype(jnp.int32)
    tail = tail.astype(jnp.int32)
    ent_t = jnp.transpose(entity_table)
    relp = jnp.pad(relation_table, ((0, 0), (0, 64)))
    tailblk = jnp.pad(ent_t[:, _TAILC0:], ((0, 0), (0, 128 - _TAILW)))
    (scratch,) = _scan_kernel(head, tail, ent_t, tailblk)
    out_h, out_r, out_t = _final_kernel(scratch, relp, relation)
    return (jnp.transpose(out_h), jnp.transpose(out_r), jnp.transpose(out_t))
